# baseline (speedup 1.0000x reference)
import functools

import jax
import jax.numpy as jnp
from jax import lax
from jax.experimental import pallas as pl
from jax.experimental.pallas import tpu as pltpu
from jax.experimental.pallas import tpu_sc as plsc

N, D, F = 65536, 512, 128
BRA = 4096
NBA = N // BRA
BR = 4096
NB = N // BR

_SC_MESH = plsc.VectorSubcoreMesh(core_axis_name="c", subcore_axis_name="s")


@functools.partial(
    pl.kernel,
    mesh=_SC_MESH,
    out_type=jax.ShapeDtypeStruct((D,), jnp.float32),
    scratch_types=[
        pltpu.VMEM((F,), jnp.int32),
        pltpu.VMEM((D,), jnp.float32),
    ],
    compiler_params=pltpu.CompilerParams(needs_layout_passes=False),
)
def _mask_sc(idx_hbm, mask_hbm, idx_v, mask_v):
    wid = lax.axis_index("s") * 2 + lax.axis_index("c")

    @pl.when(wid == 0)
    def _():
        pltpu.sync_copy(idx_hbm, idx_v)
        for i in range(D // 16):
            mask_v[pl.ds(i * 16, 16)] = jnp.zeros((16,), jnp.float32)
        ones = jnp.ones((16,), jnp.float32)
        for i in range(F // 16):
            plsc.store_scatter(mask_v, [idx_v[pl.ds(i * 16, 16)]], ones)
        pltpu.sync_copy(mask_v, mask_hbm)


def _minmax_body(x_ref, mn_ref, mx_ref):
    i = pl.program_id(0)
    x = x_ref[...].reshape(BRA // 8, 8, D)
    pmn = jnp.min(x, axis=0)
    pmx = jnp.max(x, axis=0)

    @pl.when(i == 0)
    def _():
        mn_ref[...] = pmn
        mx_ref[...] = pmx

    @pl.when(i > 0)
    def _():
        mn_ref[...] = jnp.minimum(mn_ref[...], pmn)
        mx_ref[...] = jnp.maximum(mx_ref[...], pmx)


def _apply_body(mask_ref, mn_ref, mx_ref, x_ref, o_ref):
    sel = mask_ref[...] > 0.0
    mn = jnp.min(mn_ref[...], axis=0, keepdims=True)
    mx = jnp.max(mx_ref[...], axis=0, keepdims=True)
    rs = 1.0 / (mx - mn)
    a = jnp.where(sel, rs, 1.0)
    b = jnp.where(sel, -mn * rs, 0.0)
    o_ref[...] = x_ref[...] * a + b


def kernel(inp, feature_idx):
    mn8, mx8 = pl.pallas_call(
        _minmax_body,
        grid=(NBA,),
        in_specs=[pl.BlockSpec((BRA, D), lambda i: (i, 0))],
        out_specs=[
            pl.BlockSpec((8, D), lambda i: (0, 0)),
            pl.BlockSpec((8, D), lambda i: (0, 0)),
        ],
        out_shape=[
            jax.ShapeDtypeStruct((8, D), jnp.float32),
            jax.ShapeDtypeStruct((8, D), jnp.float32),
        ],
        compiler_params=pltpu.CompilerParams(
            dimension_semantics=("arbitrary",)),
    )(inp)

    mask = _mask_sc(feature_idx.astype(jnp.int32)).reshape(1, D)

    out = pl.pallas_call(
        _apply_body,
        grid=(NB,),
        in_specs=[
            pl.BlockSpec((1, D), lambda i: (0, 0)),
            pl.BlockSpec((8, D), lambda i: (0, 0)),
            pl.BlockSpec((8, D), lambda i: (0, 0)),
            pl.BlockSpec((BR, D), lambda i: (i, 0)),
        ],
        out_specs=pl.BlockSpec((BR, D), lambda i: (i, 0)),
        out_shape=jax.ShapeDtypeStruct((N, D), jnp.float32),
        compiler_params=pltpu.CompilerParams(
            dimension_semantics=("parallel",)),
    )(mask, mn8, mx8, inp)
    return out


# speedup vs baseline: 1.1056x; 1.1056x over previous
import functools

import jax
import jax.numpy as jnp
from jax import lax
from jax.experimental import pallas as pl
from jax.experimental.pallas import tpu as pltpu
from jax.experimental.pallas import tpu_sc as plsc

N, D, F = 65536, 512, 128
BRA = 4096
NBA = N // BRA
BR = 4096
NB = N // BR

_SC_MESH = plsc.VectorSubcoreMesh(core_axis_name="c", subcore_axis_name="s")


@functools.partial(
    pl.kernel,
    mesh=_SC_MESH,
    out_type=jax.ShapeDtypeStruct((D,), jnp.float32),
    scratch_types=[
        pltpu.VMEM((F,), jnp.int32),
        pltpu.VMEM((D,), jnp.float32),
    ],
    compiler_params=pltpu.CompilerParams(needs_layout_passes=False),
)
def _mask_sc(idx_hbm, mask_hbm, idx_v, mask_v):
    wid = lax.axis_index("s") * 2 + lax.axis_index("c")

    @pl.when(wid == 0)
    def _():
        pltpu.sync_copy(idx_hbm, idx_v)
        for i in range(D // 16):
            mask_v[pl.ds(i * 16, 16)] = jnp.zeros((16,), jnp.float32)
        ones = jnp.ones((16,), jnp.float32)
        for i in range(F // 16):
            plsc.store_scatter(mask_v, [idx_v[pl.ds(i * 16, 16)]], ones)
        pltpu.sync_copy(mask_v, mask_hbm)


def _minmax_body(x_ref, mn_ref, mx_ref):
    i = pl.program_id(0)
    x = x_ref[...].reshape(BRA // 8, 8, D)
    pmn = jnp.min(x, axis=0)
    pmx = jnp.max(x, axis=0)

    @pl.when(i == 0)
    def _():
        mn_ref[...] = pmn
        mx_ref[...] = pmx

    @pl.when(i > 0)
    def _():
        mn_ref[...] = jnp.minimum(mn_ref[...], pmn)
        mx_ref[...] = jnp.maximum(mx_ref[...], pmx)


def _apply_body(mask_ref, mn_ref, mx_ref, x_ref, o_ref):
    sel = mask_ref[...] > 0.0
    mn = jnp.min(mn_ref[...], axis=0, keepdims=True)
    mx = jnp.max(mx_ref[...], axis=0, keepdims=True)
    rs = 1.0 / (mx - mn)
    a = jnp.where(sel, rs, 1.0)
    b = jnp.where(sel, -mn * rs, 0.0)
    o_ref[...] = x_ref[...] * a + b


def kernel(inp, feature_idx):
    mn8, mx8 = pl.pallas_call(
        _minmax_body,
        grid=(NBA,),
        in_specs=[pl.BlockSpec((BRA, D), lambda i: (i, 0))],
        out_specs=[
            pl.BlockSpec((8, D), lambda i: (0, 0)),
            pl.BlockSpec((8, D), lambda i: (0, 0)),
        ],
        out_shape=[
            jax.ShapeDtypeStruct((8, D), jnp.float32),
            jax.ShapeDtypeStruct((8, D), jnp.float32),
        ],
        compiler_params=pltpu.CompilerParams(
            dimension_semantics=("arbitrary",)),
    )(inp)

    mask = jnp.zeros((D,), jnp.float32).at[feature_idx].set(1.0).reshape(1, D)

    out = pl.pallas_call(
        _apply_body,
        grid=(NB,),
        in_specs=[
            pl.BlockSpec((1, D), lambda i: (0, 0)),
            pl.BlockSpec((8, D), lambda i: (0, 0)),
            pl.BlockSpec((8, D), lambda i: (0, 0)),
            pl.BlockSpec((BR, D), lambda i: (i, 0)),
        ],
        out_specs=pl.BlockSpec((BR, D), lambda i: (i, 0)),
        out_shape=jax.ShapeDtypeStruct((N, D), jnp.float32),
        compiler_params=pltpu.CompilerParams(
            dimension_semantics=("parallel",)),
    )(mask, mn8, mx8, inp)
    return out


# speedup vs baseline: 1.1763x; 1.0640x over previous
import jax
import jax.numpy as jnp
from jax.experimental import pallas as pl
from jax.experimental.pallas import tpu as pltpu

N, D, F = 65536, 512, 128
BR = 2048
NB = N // BR
K = 9


def _fused_body(idx_ref, x_ref, o_ref, stash_ref, mn_ref, mx_ref):
    s = pl.program_id(0)

    @pl.when(s < NB)
    def _phase_a():
        x = x_ref[...]
        xr = x.reshape(BR // 8, 8, D)
        pmn = jnp.min(xr, axis=0)
        pmx = jnp.max(xr, axis=0)

        @pl.when(s == 0)
        def _():
            mn_ref[...] = pmn
            mx_ref[...] = pmx

        @pl.when(s > 0)
        def _():
            mn_ref[...] = jnp.minimum(mn_ref[...], pmn)
            mx_ref[...] = jnp.maximum(mx_ref[...], pmx)

        @pl.when(s >= NB - K)
        def _():
            stash_ref[jnp.maximum(s - (NB - K), 0)] = x

    @pl.when(s >= NB)
    def _phase_b():
        j = s - NB
        ci = jax.lax.broadcasted_iota(jnp.int32, (F, D), 1)
        sel = jnp.any(ci == idx_ref[...], axis=0, keepdims=True)
        mn = jnp.min(mn_ref[...], axis=0, keepdims=True)
        mx = jnp.max(mx_ref[...], axis=0, keepdims=True)
        rs = 1.0 / (mx - mn)
        a = jnp.where(sel, rs, 1.0)
        b = jnp.where(sel, -mn * rs, 0.0)

        @pl.when(j < NB - K)
        def _():
            o_ref[...] = x_ref[...] * a + b

        @pl.when(j >= NB - K)
        def _():
            o_ref[...] = stash_ref[jnp.maximum(j - (NB - K), 0)] * a + b


def _x_index(s):
    j = s - NB
    return (jnp.where(s < NB, s, jnp.minimum(j, NB - K - 1)), 0)


def _o_index(s):
    return (jnp.where(s < NB, 0, s - NB), 0)


def kernel(inp, feature_idx):
    idx2d = feature_idx.astype(jnp.int32).reshape(F, 1)
    out = pl.pallas_call(
        _fused_body,
        grid=(2 * NB,),
        in_specs=[
            pl.BlockSpec((F, 1), lambda s: (0, 0)),
            pl.BlockSpec((BR, D), _x_index),
        ],
        out_specs=pl.BlockSpec((BR, D), _o_index),
        out_shape=jax.ShapeDtypeStruct((N, D), jnp.float32),
        scratch_shapes=[
            pltpu.VMEM((K, BR, D), jnp.float32),
            pltpu.VMEM((8, D), jnp.float32),
            pltpu.VMEM((8, D), jnp.float32),
        ],
        compiler_params=pltpu.CompilerParams(
            dimension_semantics=("arbitrary",)),
    )(idx2d, inp)
    return out
